# Initial kernel scaffold; baseline (speedup 1.0000x reference)
import jax
import jax.numpy as jnp
from jax.experimental import pallas as pl


def kernel(x, pe):
    raise NotImplementedError("write your pallas kernel here")


# speedup vs baseline: 2.2486x; 2.2486x over previous
import functools

import jax
import jax.numpy as jnp
from jax import lax
from jax.experimental import pallas as pl
from jax.experimental.pallas import tpu as pltpu
from jax.experimental.pallas import tpu_sc as plsc

NC = 2
NS = 16
NW = NC * NS
D = 1024
CHUNK = 32
NBUF = 2


def _gather_body(x_hbm, pe_hbm, out_hbm, idx_v, *rest):
    nch = idx_v.shape[0]
    bufs = rest[:NBUF]
    gsems = rest[NBUF:2 * NBUF]
    ssems = rest[2 * NBUF:3 * NBUF]

    cid = lax.axis_index("c")
    sid = lax.axis_index("s")
    wid = sid * NC + cid

    pltpu.sync_copy(x_hbm.at[wid], idx_v)

    for b in range(NBUF):
        pltpu.async_copy(pe_hbm.at[idx_v.at[b]], bufs[b], gsems[b])

    def outer(i, carry):
        for b in range(NBUF):
            g = i * NBUF + b
            pltpu.make_async_copy(pe_hbm.at[pl.ds(0, CHUNK)], bufs[b],
                                  gsems[b]).wait()
            pltpu.async_copy(bufs[b], out_hbm.at[wid, g], ssems[b])
        for b in range(NBUF):
            g = i * NBUF + b + NBUF

            @pl.when(g < nch)
            def _():
                pltpu.make_async_copy(bufs[b], out_hbm.at[wid, 0],
                                      ssems[b]).wait()
                pltpu.async_copy(pe_hbm.at[idx_v.at[g]], bufs[b], gsems[b])

        return carry

    lax.fori_loop(0, nch // NBUF, outer, 0)

    for b in range(NBUF):
        pltpu.make_async_copy(bufs[b], out_hbm.at[wid, 0], ssems[b]).wait()


@jax.jit
def _sc_gather(x_resh, pe):
    nch = x_resh.shape[1]
    mesh = plsc.VectorSubcoreMesh(core_axis_name="c", subcore_axis_name="s")
    scratch = (
        [pltpu.VMEM((nch, CHUNK), jnp.int32)]
        + [pltpu.VMEM((CHUNK, D), jnp.float32) for _ in range(NBUF)]
        + [pltpu.SemaphoreType.DMA for _ in range(2 * NBUF)]
    )
    run = pl.kernel(
        _gather_body,
        out_type=jax.ShapeDtypeStruct((NW, nch, CHUNK, D), jnp.float32),
        mesh=mesh,
        scratch_types=scratch,
    )
    return run(x_resh, pe)


def kernel(x, pe):
    B, S = x.shape
    total = B * S
    per_w = total // NW
    nch = per_w // CHUNK
    x_resh = x.reshape(NW, nch, CHUNK)
    out = _sc_gather(x_resh, pe)
    return out.reshape(B, S, D)


# speedup vs baseline: 2.2914x; 1.0191x over previous
import functools

import jax
import jax.numpy as jnp
from jax import lax
from jax.experimental import pallas as pl
from jax.experimental.pallas import tpu as pltpu
from jax.experimental.pallas import tpu_sc as plsc

NC = 2
NS = 16
NW = NC * NS
D = 1024
CHUNK = 32
NBUF = 2


def _gather_body(x_hbm, pe_hbm, out_hbm, idx_v, *rest):
    nch = idx_v.shape[0]
    bufs = rest[:NBUF]
    gsems = rest[NBUF:2 * NBUF]
    ssems = rest[2 * NBUF:3 * NBUF]

    cid = lax.axis_index("c")
    sid = lax.axis_index("s")
    wid = sid * NC + cid

    pltpu.sync_copy(x_hbm.at[wid], idx_v)

    pltpu.async_copy(pe_hbm.at[idx_v.at[0]], bufs[0], gsems[0])

    def outer(i, carry):
        for b in range(NBUF):
            g = i * NBUF + b
            nb = (b + 1) % NBUF
            pltpu.make_async_copy(pe_hbm.at[pl.ds(0, CHUNK)], bufs[b],
                                  gsems[b]).wait()
            pltpu.async_copy(bufs[b], out_hbm.at[wid, g], ssems[b])
            if b == 0:
                @pl.when(i >= 1)
                def _():
                    pltpu.make_async_copy(bufs[nb], out_hbm.at[wid, 0],
                                          ssems[nb]).wait()

                pltpu.async_copy(pe_hbm.at[idx_v.at[g + 1]], bufs[nb],
                                 gsems[nb])
            else:
                @pl.when(g + 1 < nch)
                def _():
                    pltpu.make_async_copy(bufs[nb], out_hbm.at[wid, 0],
                                          ssems[nb]).wait()
                    pltpu.async_copy(pe_hbm.at[idx_v.at[g + 1]], bufs[nb],
                                     gsems[nb])

        return carry

    lax.fori_loop(0, nch // NBUF, outer, 0)

    for b in range(NBUF):
        pltpu.make_async_copy(bufs[b], out_hbm.at[wid, 0], ssems[b]).wait()


@jax.jit
def _sc_gather(x_resh, pe):
    nch = x_resh.shape[1]
    mesh = plsc.VectorSubcoreMesh(core_axis_name="c", subcore_axis_name="s")
    scratch = (
        [pltpu.VMEM((nch, CHUNK), jnp.int32)]
        + [pltpu.VMEM((CHUNK, D), jnp.float32) for _ in range(NBUF)]
        + [pltpu.SemaphoreType.DMA for _ in range(2 * NBUF)]
    )
    run = pl.kernel(
        _gather_body,
        out_type=jax.ShapeDtypeStruct((NW, nch, CHUNK, D), jnp.float32),
        mesh=mesh,
        scratch_types=scratch,
    )
    return run(x_resh, pe)


def kernel(x, pe):
    B, S = x.shape
    total = B * S
    per_w = total // NW
    nch = per_w // CHUNK
    x_resh = x.reshape(NW, nch, CHUNK)
    out = _sc_gather(x_resh, pe)
    return out.reshape(B, S, D)
